# initial kernel scaffold (unmeasured)
import jax
import jax.numpy as jnp
from jax import lax
from jax.experimental import pallas as pl
from jax.experimental.pallas import tpu as pltpu

B, H, D = 16, 16, 64
SCALE = D ** -0.5
NY = 4


def _attn_body(q_ref, k_ref, v_ref, o_ref, m_ref, l_ref):
    q = q_ref[0, 0]
    k = k_ref[0, :, 0, :]
    v = v_ref[0, :, 0, :]
    s = jnp.sum(k * q, axis=1, keepdims=True) * SCALE
    m = jnp.max(s)
    p = jnp.exp(s - m)
    l = jnp.sum(p)
    o = jnp.sum(p * v, axis=0, keepdims=True)
    o_ref[0] = o
    m_ref[...] = jnp.reshape(m, (1, 1))
    l_ref[...] = jnp.reshape(l, (1, 1))


def _attn(Q, K, V):
    S = K.shape[1]
    return pl.pallas_call(
        _attn_body,
        grid=(B, H),
        in_specs=[
            pl.BlockSpec((1, 1, 1, D), lambda b, h: (b, 0, h, 0)),
            pl.BlockSpec((1, S, 1, D), lambda b, h: (b, 0, h, 0)),
            pl.BlockSpec((1, S, 1, D), lambda b, h: (b, 0, h, 0)),
        ],
        out_specs=[
            pl.BlockSpec((1, 1, D), lambda b, h: (b, h, 0)),
            pl.BlockSpec((1, 1), lambda b, h: (b, h)),
            pl.BlockSpec((1, 1), lambda b, h: (b, h)),
        ],
        out_shape=[
            jax.ShapeDtypeStruct((B, H, D), jnp.float32),
            jax.ShapeDtypeStruct((B, H), jnp.float32),
            jax.ShapeDtypeStruct((B, H), jnp.float32),
        ],
    )(Q, K, V)


def _combine_body(o_ref, m_ref, l_ref, out_ref,
                  o_comm, m_comm, l_comm, send_sems, recv_sems):
    my_x = lax.axis_index("x")
    my_y = lax.axis_index("y")
    my_z = lax.axis_index("z")

    barrier = pltpu.get_barrier_semaphore()
    for dy in range(1, NY):
        pl.semaphore_signal(
            barrier, inc=1,
            device_id=(my_x, (my_y + dy) % NY, my_z),
            device_id_type=pl.DeviceIdType.MESH,
        )
    pl.semaphore_wait(barrier, NY - 1)

    for me in range(NY):
        @pl.when(my_y == me)
        def _(me=me):
            o_comm[me] = o_ref[...]
            m_comm[me] = m_ref[...]
            l_comm[me] = l_ref[...]
            comms = (o_comm, m_comm, l_comm)
            rdmas = []
            for i, dy in enumerate(range(1, NY)):
                tgt = (me + dy) % NY
                for kind, comm in enumerate(comms):
                    rdma = pltpu.make_async_remote_copy(
                        src_ref=comm.at[me],
                        dst_ref=comm.at[me],
                        send_sem=send_sems.at[i, kind],
                        recv_sem=recv_sems.at[me, kind],
                        device_id=(my_x, tgt, my_z),
                        device_id_type=pl.DeviceIdType.MESH,
                    )
                    rdma.start()
                    rdmas.append(rdma)
            for j in range(NY):
                if j == me:
                    continue
                for kind, comm in enumerate(comms):
                    recv = pltpu.make_async_remote_copy(
                        src_ref=comm.at[j],
                        dst_ref=comm.at[j],
                        send_sem=send_sems.at[0, kind],
                        recv_sem=recv_sems.at[j, kind],
                        device_id=(my_x, me, my_z),
                        device_id_type=pl.DeviceIdType.MESH,
                    )
                    recv.wait_recv()
            for rdma in rdmas:
                rdma.wait_send()

    M = m_comm[0]
    for j in range(1, NY):
        M = jnp.maximum(M, m_comm[j])
    acc_l = jnp.zeros((B, H), jnp.float32)
    acc_o = jnp.zeros((B, H, D), jnp.float32)
    for j in range(NY):
        w = jnp.exp(m_comm[j] - M)
        acc_l = acc_l + l_comm[j] * w
        acc_o = acc_o + o_comm[j] * w[:, :, None]
    out_ref[...] = acc_o / acc_l[:, :, None]


def _combine(o, m, l):
    return pl.pallas_call(
        _combine_body,
        out_shape=jax.ShapeDtypeStruct((B, H, D), jnp.float32),
        in_specs=[pl.BlockSpec(memory_space=pltpu.VMEM)] * 3,
        out_specs=pl.BlockSpec(memory_space=pltpu.VMEM),
        scratch_shapes=[
            pltpu.VMEM((NY, B, H, D), jnp.float32),
            pltpu.VMEM((NY, B, H), jnp.float32),
            pltpu.VMEM((NY, B, H), jnp.float32),
            pltpu.SemaphoreType.DMA((NY - 1, 3)),
            pltpu.SemaphoreType.DMA((NY, 3)),
        ],
        compiler_params=pltpu.CompilerParams(collective_id=0),
    )(o, m, l)


def kernel(Q, K, V):
    o, m, l = _attn(Q, K, V)
    out = _combine(o, m, l)
    return out.reshape(B, 1, H, D)


# baseline (device time: 347636 ns/iter reference)
import jax
import jax.numpy as jnp
from jax import lax
from jax.experimental import pallas as pl
from jax.experimental.pallas import tpu as pltpu

B, H, D = 16, 16, 64
HC = 8
SCALE = D ** -0.5
NY = 4


def _attn_body(q_ref, k_ref, v_ref, o_ref, m_ref, l_ref):
    q = q_ref[0, 0]
    k = k_ref[0]
    v = v_ref[0]
    s = jnp.sum(k * q[None], axis=2) * SCALE
    m = jnp.max(s, axis=0, keepdims=True)
    p = jnp.exp(s - m)
    l = jnp.sum(p, axis=0, keepdims=True)
    o = jnp.sum(p[:, :, None] * v, axis=0)
    o_ref[0] = o
    m_ref[0] = m.reshape(HC, 1)
    l_ref[0] = l.reshape(HC, 1)


def _attn(Q, K, V):
    S = K.shape[1]
    return pl.pallas_call(
        _attn_body,
        grid=(B, H // HC),
        in_specs=[
            pl.BlockSpec((1, 1, HC, D), lambda b, h: (b, 0, h, 0)),
            pl.BlockSpec((1, S, HC, D), lambda b, h: (b, 0, h, 0)),
            pl.BlockSpec((1, S, HC, D), lambda b, h: (b, 0, h, 0)),
        ],
        out_specs=[
            pl.BlockSpec((1, HC, D), lambda b, h: (b, h, 0)),
            pl.BlockSpec((1, HC, 1), lambda b, h: (b, h, 0)),
            pl.BlockSpec((1, HC, 1), lambda b, h: (b, h, 0)),
        ],
        out_shape=[
            jax.ShapeDtypeStruct((B, H, D), jnp.float32),
            jax.ShapeDtypeStruct((B, H, 1), jnp.float32),
            jax.ShapeDtypeStruct((B, H, 1), jnp.float32),
        ],
    )(Q, K, V)


def _combine_body(o_ref, m_ref, l_ref, out_ref,
                  o_comm, m_comm, l_comm, send_sems, recv_sems):
    my_x = lax.axis_index("x")
    my_y = lax.axis_index("y")
    my_z = lax.axis_index("z")

    barrier = pltpu.get_barrier_semaphore()
    for dy in range(1, NY):
        pl.semaphore_signal(
            barrier, inc=1,
            device_id=(my_x, (my_y + dy) % NY, my_z),
            device_id_type=pl.DeviceIdType.MESH,
        )
    pl.semaphore_wait(barrier, NY - 1)

    for me in range(NY):
        @pl.when(my_y == me)
        def _(me=me):
            o_comm[me] = o_ref[...]
            m_comm[me] = m_ref[...]
            l_comm[me] = l_ref[...]
            comms = (o_comm, m_comm, l_comm)
            rdmas = []
            for i, dy in enumerate(range(1, NY)):
                tgt = (me + dy) % NY
                for kind, comm in enumerate(comms):
                    rdma = pltpu.make_async_remote_copy(
                        src_ref=comm.at[me],
                        dst_ref=comm.at[me],
                        send_sem=send_sems.at[i, kind],
                        recv_sem=recv_sems.at[me, kind],
                        device_id=(my_x, tgt, my_z),
                        device_id_type=pl.DeviceIdType.MESH,
                    )
                    rdma.start()
                    rdmas.append(rdma)
            for j in range(NY):
                if j == me:
                    continue
                for kind, comm in enumerate(comms):
                    recv = pltpu.make_async_remote_copy(
                        src_ref=comm.at[j],
                        dst_ref=comm.at[j],
                        send_sem=send_sems.at[0, kind],
                        recv_sem=recv_sems.at[j, kind],
                        device_id=(my_x, me, my_z),
                        device_id_type=pl.DeviceIdType.MESH,
                    )
                    recv.wait_recv()
            for rdma in rdmas:
                rdma.wait_send()

    M = m_comm[0]
    for j in range(1, NY):
        M = jnp.maximum(M, m_comm[j])
    acc_l = jnp.zeros((B, H, 1), jnp.float32)
    acc_o = jnp.zeros((B, H, D), jnp.float32)
    for j in range(NY):
        w = jnp.exp(m_comm[j] - M)
        acc_l = acc_l + l_comm[j] * w
        acc_o = acc_o + o_comm[j] * w
    out_ref[...] = acc_o / acc_l


def _combine(o, m, l):
    return pl.pallas_call(
        _combine_body,
        out_shape=jax.ShapeDtypeStruct((B, H, D), jnp.float32),
        in_specs=[pl.BlockSpec(memory_space=pltpu.VMEM)] * 3,
        out_specs=pl.BlockSpec(memory_space=pltpu.VMEM),
        scratch_shapes=[
            pltpu.VMEM((NY, B, H, D), jnp.float32),
            pltpu.VMEM((NY, B, H, 1), jnp.float32),
            pltpu.VMEM((NY, B, H, 1), jnp.float32),
            pltpu.SemaphoreType.DMA((NY - 1, 3)),
            pltpu.SemaphoreType.DMA((NY, 3)),
        ],
        compiler_params=pltpu.CompilerParams(collective_id=0),
    )(o, m, l)


def kernel(Q, K, V):
    o, m, l = _attn(Q, K, V)
    out = _combine(o, m, l)
    return out.reshape(B, 1, H, D)


# device time: 189131 ns/iter; 1.8381x vs baseline; 1.8381x over previous
import jax
import jax.numpy as jnp
from jax import lax
from jax.experimental import pallas as pl
from jax.experimental.pallas import tpu as pltpu

B, H, D = 16, 16, 64
HD = H * D
SCALE = D ** -0.5
NY = 4


def _attn_body(q_ref, k_ref, v_ref, out_ref):
    q = q_ref[0, 0]
    kf = k_ref[0]
    vf = v_ref[0]
    r = lax.broadcasted_iota(jnp.int32, (H, HD), 0)
    c = lax.broadcasted_iota(jnp.int32, (H, HD), 1)
    e = (c // D == r).astype(jnp.float32)
    qm = jnp.concatenate([q] * H, axis=1) * e
    s = lax.dot_general(
        qm, kf, (((1,), (1,)), ((), ())),
        preferred_element_type=jnp.float32,
    ) * SCALE
    m = jnp.max(s, axis=1, keepdims=True)
    p = jnp.exp(s - m)
    l = jnp.sum(p, axis=1, keepdims=True)
    o_full = lax.dot_general(
        p, vf, (((1,), (0,)), ((), ())),
        preferred_element_type=jnp.float32,
    )
    o_vec = jnp.sum(o_full * e, axis=0, keepdims=True)
    m_flat = jnp.sum(m * e, axis=0, keepdims=True)
    l_flat = jnp.sum(l * e, axis=0, keepdims=True)
    out_ref[0] = jnp.concatenate([o_vec, m_flat, l_flat], axis=0)


def _attn(Q, K, V):
    S = K.shape[1]
    Kf = K.reshape(B, S, HD)
    Vf = V.reshape(B, S, HD)
    return pl.pallas_call(
        _attn_body,
        grid=(B,),
        in_specs=[
            pl.BlockSpec((1, 1, H, D), lambda b: (b, 0, 0, 0)),
            pl.BlockSpec((1, S, HD), lambda b: (b, 0, 0)),
            pl.BlockSpec((1, S, HD), lambda b: (b, 0, 0)),
        ],
        out_specs=pl.BlockSpec((1, 3, HD), lambda b: (b, 0, 0)),
        out_shape=jax.ShapeDtypeStruct((B, 3, HD), jnp.float32),
        compiler_params=pltpu.CompilerParams(
            vmem_limit_bytes=100 * 1024 * 1024,
        ),
    )(Q, Kf, Vf)


def _combine_body(p_ref, out_ref, comm, send_sems, recv_sems):
    my_x = lax.axis_index("x")
    my_y = lax.axis_index("y")
    my_z = lax.axis_index("z")

    barrier = pltpu.get_barrier_semaphore()
    for dy in range(1, NY):
        pl.semaphore_signal(
            barrier, inc=1,
            device_id=(my_x, (my_y + dy) % NY, my_z),
            device_id_type=pl.DeviceIdType.MESH,
        )
    pl.semaphore_wait(barrier, NY - 1)

    for me in range(NY):
        @pl.when(my_y == me)
        def _(me=me):
            comm[me] = p_ref[...]
            rdmas = []
            for i, dy in enumerate(range(1, NY)):
                tgt = (me + dy) % NY
                rdma = pltpu.make_async_remote_copy(
                    src_ref=comm.at[me],
                    dst_ref=comm.at[me],
                    send_sem=send_sems.at[i],
                    recv_sem=recv_sems.at[me],
                    device_id=(my_x, tgt, my_z),
                    device_id_type=pl.DeviceIdType.MESH,
                )
                rdma.start()
                rdmas.append(rdma)
            for j in range(NY):
                if j == me:
                    continue
                recv = pltpu.make_async_remote_copy(
                    src_ref=comm.at[j],
                    dst_ref=comm.at[j],
                    send_sem=send_sems.at[0],
                    recv_sem=recv_sems.at[j],
                    device_id=(my_x, me, my_z),
                    device_id_type=pl.DeviceIdType.MESH,
                )
                recv.wait_recv()
            for rdma in rdmas:
                rdma.wait_send()

    parts = [comm[j] for j in range(NY)]
    ms = [pt[:, 1:2, :] for pt in parts]
    M = ms[0]
    for j in range(1, NY):
        M = jnp.maximum(M, ms[j])
    acc_o = jnp.zeros((B, 1, HD), jnp.float32)
    acc_l = jnp.zeros((B, 1, HD), jnp.float32)
    for j in range(NY):
        w = jnp.exp(ms[j] - M)
        acc_o = acc_o + parts[j][:, 0:1, :] * w
        acc_l = acc_l + parts[j][:, 2:3, :] * w
    out_ref[...] = acc_o / acc_l


def _combine(packed):
    return pl.pallas_call(
        _combine_body,
        out_shape=jax.ShapeDtypeStruct((B, 1, HD), jnp.float32),
        in_specs=[pl.BlockSpec(memory_space=pltpu.VMEM)],
        out_specs=pl.BlockSpec(memory_space=pltpu.VMEM),
        scratch_shapes=[
            pltpu.VMEM((NY, B, 3, HD), jnp.float32),
            pltpu.SemaphoreType.DMA((NY - 1,)),
            pltpu.SemaphoreType.DMA((NY,)),
        ],
        compiler_params=pltpu.CompilerParams(collective_id=0),
    )(packed)


def kernel(Q, K, V):
    packed = _attn(Q, K, V)
    out = _combine(packed)
    return out.reshape(B, 1, H, D)
